# baseline (speedup 1.0000x reference)
import jax
import jax.numpy as jnp
from jax import lax
from jax.experimental import pallas as pl
from jax.experimental.pallas import tpu as pltpu
from jax.experimental.pallas import tpu_sc as plsc

ROWS = 16384
COLS = 200
DIM = 128

SC_ROWS = 4096
SC_FLAT = SC_ROWS * COLS
_NC = 2
_NS = 16
_NW = _NC * _NS
_PER_W = SC_FLAT // _NW
_CHUNK = 400
_NCHUNK = _PER_W // _CHUNK


def _sc_body(mask_hbm, emb_hbm, out_hbm, table_s,
             idx0, idx1, rows0, rows1,
             sem_g0, sem_g1, sem_w0, sem_w1):
    sid = lax.axis_index("s")
    wid = sid * _NC + lax.axis_index("c")
    base = wid * _PER_W

    @pl.when(sid == 0)
    def _():
        pltpu.sync_copy(emb_hbm, table_s)

    plsc.subcore_barrier()

    idx_v = (idx0, idx1)
    rows_v = (rows0, rows1)
    sem_g = (sem_g0, sem_g1)
    sem_w = (sem_w0, sem_w1)

    for b in range(2):
        pltpu.sync_copy(mask_hbm.at[pl.ds(base + b * _CHUNK, _CHUNK)],
                        idx_v[b])
        pltpu.async_copy(table_s.at[idx_v[b]], rows_v[b], sem_g[b])

    def pair(t, carry):
        for b in range(2):
            g = 2 * t + b
            off = base + g * _CHUNK
            pltpu.make_async_copy(table_s.at[idx_v[b]], rows_v[b],
                                  sem_g[b]).wait()
            wr = pltpu.async_copy(rows_v[b], out_hbm.at[pl.ds(off, _CHUNK)],
                                  sem_w[b])

            @pl.when(g + 2 < _NCHUNK)
            def _():
                pltpu.sync_copy(
                    mask_hbm.at[pl.ds(off + 2 * _CHUNK, _CHUNK)], idx_v[b])

            wr.wait()

            @pl.when(g + 2 < _NCHUNK)
            def _():
                pltpu.async_copy(table_s.at[idx_v[b]], rows_v[b], sem_g[b])
        return carry

    lax.fori_loop(0, _NCHUNK // 2, pair, 0)


TC_ROWS = ROWS - SC_ROWS
BLOCK_ROWS = 256


def _tc_kernel(mask_ref, emb_ref, out_ref):
    m = mask_ref[...]
    e0 = emb_ref[0, :]
    e1 = emb_ref[1, :]
    out_ref[...] = jnp.where((m[:, :, None] != 0), e1[None, None, :],
                             e0[None, None, :])


def kernel(mask01, emb):
    mask_sc = mask01[:SC_ROWS].reshape(SC_FLAT)
    mesh = plsc.VectorSubcoreMesh(core_axis_name="c", subcore_axis_name="s")
    sc_call = pl.kernel(
        _sc_body,
        out_type=jax.ShapeDtypeStruct((SC_FLAT, DIM), jnp.float32),
        mesh=mesh,
        scratch_types=[
            pltpu.VMEM_SHARED((2, DIM), jnp.float32),
            pltpu.VMEM((_CHUNK,), jnp.int32),
            pltpu.VMEM((_CHUNK,), jnp.int32),
            pltpu.VMEM((_CHUNK, DIM), jnp.float32),
            pltpu.VMEM((_CHUNK, DIM), jnp.float32),
            pltpu.SemaphoreType.DMA,
            pltpu.SemaphoreType.DMA,
            pltpu.SemaphoreType.DMA,
            pltpu.SemaphoreType.DMA,
        ],
    )
    out_sc = sc_call(mask_sc, emb).reshape(SC_ROWS, COLS, DIM)

    out_tc = pl.pallas_call(
        _tc_kernel,
        grid=(TC_ROWS // BLOCK_ROWS,),
        in_specs=[
            pl.BlockSpec((BLOCK_ROWS, COLS), lambda i: (i, 0)),
            pl.BlockSpec((2, DIM), lambda i: (0, 0)),
        ],
        out_specs=pl.BlockSpec((BLOCK_ROWS, COLS, DIM), lambda i: (i, 0, 0)),
        out_shape=jax.ShapeDtypeStruct((TC_ROWS, COLS, DIM), jnp.float32),
    )(mask01[SC_ROWS:], emb)

    return jnp.concatenate([out_sc, out_tc], axis=0)


# speedup vs baseline: 1.6526x; 1.6526x over previous
import jax
import jax.numpy as jnp
from jax import lax
from jax.experimental import pallas as pl
from jax.experimental.pallas import tpu as pltpu
from jax.experimental.pallas import tpu_sc as plsc

ROWS = 16384
COLS = 200
DIM = 128
FLAT = ROWS * COLS

_NC = 2
_NS = 16
_NW = _NC * _NS
_PER_W = FLAT // _NW
_NBUF = 4
_CHUNK = 200
_NCHUNK = _PER_W // _CHUNK


def _sc_body(mask_hbm, emb_hbm, out_hbm, table_s, idx_v, rows_v,
             sem_g, sem_w):
    sid = lax.axis_index("s")
    wid = sid * _NC + lax.axis_index("c")
    base = wid * _PER_W

    @pl.when(sid == 0)
    def _():
        pltpu.sync_copy(emb_hbm, table_s)

    plsc.subcore_barrier()

    for b in range(_NBUF):
        pltpu.sync_copy(mask_hbm.at[pl.ds(base + b * _CHUNK, _CHUNK)],
                        idx_v[b])
        pltpu.async_copy(table_s.at[idx_v[b]], rows_v[b], sem_g[b])

    def turn(t, carry):
        for b in range(_NBUF):
            g = _NBUF * t + b
            off = base + g * _CHUNK
            pltpu.make_async_copy(table_s.at[idx_v[b]], rows_v[b],
                                  sem_g[b]).wait()
            wr = pltpu.async_copy(rows_v[b], out_hbm.at[pl.ds(off, _CHUNK)],
                                  sem_w[b])

            @pl.when(g + _NBUF < _NCHUNK)
            def _():
                pltpu.sync_copy(
                    mask_hbm.at[pl.ds(off + _NBUF * _CHUNK, _CHUNK)],
                    idx_v[b])

            wr.wait()

            @pl.when(g + _NBUF < _NCHUNK)
            def _():
                pltpu.async_copy(table_s.at[idx_v[b]], rows_v[b], sem_g[b])
        return carry

    lax.fori_loop(0, _NCHUNK // _NBUF, turn, 0)


def kernel(mask01, emb):
    mask_flat = mask01.reshape(FLAT)
    mesh = plsc.VectorSubcoreMesh(core_axis_name="c", subcore_axis_name="s")
    k = pl.kernel(
        _sc_body,
        out_type=jax.ShapeDtypeStruct((FLAT, DIM), jnp.float32),
        mesh=mesh,
        scratch_types=[
            pltpu.VMEM_SHARED((2, DIM), jnp.float32),
            [pltpu.VMEM((_CHUNK,), jnp.int32) for _ in range(_NBUF)],
            [pltpu.VMEM((_CHUNK, DIM), jnp.float32) for _ in range(_NBUF)],
            [pltpu.SemaphoreType.DMA for _ in range(_NBUF)],
            [pltpu.SemaphoreType.DMA for _ in range(_NBUF)],
        ],
    )
    out = k(mask_flat, emb)
    return out.reshape(ROWS, COLS, DIM)
